# Initial kernel scaffold; baseline (speedup 1.0000x reference)
import jax
import jax.numpy as jnp
from jax.experimental import pallas as pl


def kernel(input_ids, attention_mask, edge_idx, edge_type, basis, comp, root, rgcn_bias, attn_a, attn_b, out_bias):
    raise NotImplementedError("write your pallas kernel here")


# speedup vs baseline: 1.6434x; 1.6434x over previous
import functools

import jax
import jax.numpy as jnp
from jax import lax
from jax.experimental import pallas as pl
from jax.experimental.pallas import tpu as pltpu
from jax.experimental.pallas import tpu_sc as plsc

N = 10000
R = 12
D = 128
NB = 8
E = 320000
B = 64
L = 50

NC = 2
NS = 16
NW = NC * NS

C = 128
T = 10240
E_PAD = NW * T
NCHUNK = T // C
PER_CORE = E_PAD // NC

AGG_ROWS = N + 240
KEYS = AGG_ROWS * R
KEY_SLICE = KEYS // NS
AGG_SLICE = AGG_ROWS // NS

H_PAD = 3328
H_PER_W = H_PAD // NW

_mesh = plsc.VectorSubcoreMesh(core_axis_name="c", subcore_axis_name="s")


@functools.partial(
    pl.kernel,
    mesh=_mesh,
    out_type=jax.ShapeDtypeStruct((NC, KEYS), jnp.float32),
    scratch_types=[
        pltpu.VMEM((C,), jnp.int32),
        pltpu.VMEM((C,), jnp.int32),
        pltpu.VMEM((C,), jnp.int32),
        pltpu.VMEM((C,), jnp.float32),
        pltpu.VMEM((KEY_SLICE,), jnp.float32),
        pltpu.VMEM_SHARED((KEYS,), jnp.float32),
        pltpu.SemaphoreType.DMA,
    ],
)
def _sc_count(dst_hbm, et_hbm, out_hbm, dstv, etv, keyv, onesv, zerov, cnt_sh, sem):
    c = lax.axis_index("c")
    s = lax.axis_index("s")

    def _fill(i, _):
        zerov[pl.ds(i * 16, 16)] = jnp.zeros((16,), jnp.float32)
        return 0
    lax.fori_loop(0, KEY_SLICE // 16, _fill, 0)
    for j in range(C // 16):
        onesv[pl.ds(j * 16, 16)] = jnp.ones((16,), jnp.float32)
    pltpu.sync_copy(zerov, cnt_sh.at[pl.ds(s * KEY_SLICE, KEY_SLICE)])
    plsc.subcore_barrier()

    base = c * PER_CORE + s * T

    def _chunk(i, _):
        off = base + i * C
        pltpu.sync_copy(dst_hbm.at[pl.ds(off, C)], dstv)
        pltpu.sync_copy(et_hbm.at[pl.ds(off, C)], etv)
        for j in range(C // 16):
            sl = pl.ds(j * 16, 16)
            keyv[sl] = dstv[sl] * R + etv[sl]
        pltpu.sync_copy(onesv, cnt_sh.at[keyv], add=True)
        return 0
    lax.fori_loop(0, NCHUNK, _chunk, 0)

    plsc.subcore_barrier()
    sl = pl.ds(s * KEY_SLICE, KEY_SLICE)
    pltpu.sync_copy(cnt_sh.at[sl], out_hbm.at[c, sl])


@functools.partial(
    pl.kernel,
    mesh=_mesh,
    out_type=jax.ShapeDtypeStruct((NC, AGG_ROWS, D), jnp.float32),
    scratch_types=[
        pltpu.VMEM((C,), jnp.int32),
        pltpu.VMEM((C,), jnp.int32),
        pltpu.VMEM((C,), jnp.int32),
        pltpu.VMEM((C,), jnp.int32),
        pltpu.VMEM((C,), jnp.int32),
        pltpu.VMEM((C,), jnp.float32),
        pltpu.VMEM((C, D), jnp.float32),
        pltpu.VMEM_SHARED((AGG_ROWS, D), jnp.float32),
        pltpu.SemaphoreType.DMA,
        pltpu.SemaphoreType.DMA,
    ],
)
def _sc_main(src_hbm, dst_hbm, et_hbm, w_hbm, norm_hbm, out_hbm,
             srcv, dstv, etv, widxv, keyv, normv, rows, agg_sh, sem_w, sem_n):
    c = lax.axis_index("c")
    s = lax.axis_index("s")

    def _zrow(k, _):
        for j in range(D // 16):
            rows[k, pl.ds(j * 16, 16)] = jnp.zeros((16,), jnp.float32)
        return 0
    lax.fori_loop(0, C, _zrow, 0)

    def _zcopy(k, _):
        pltpu.sync_copy(rows, agg_sh.at[pl.ds(s * AGG_SLICE + k * C, C)])
        return 0
    lax.fori_loop(0, AGG_SLICE // C, _zcopy, 0)
    plsc.subcore_barrier()

    base = c * PER_CORE + s * T

    def _chunk(i, _):
        off = base + i * C
        pltpu.sync_copy(src_hbm.at[pl.ds(off, C)], srcv)
        pltpu.sync_copy(dst_hbm.at[pl.ds(off, C)], dstv)
        pltpu.sync_copy(et_hbm.at[pl.ds(off, C)], etv)
        for j in range(C // 16):
            sl = pl.ds(j * 16, 16)
            widxv[sl] = etv[sl] * N + srcv[sl]
            keyv[sl] = dstv[sl] * R + etv[sl]
        cp_w = pltpu.async_copy(w_hbm.at[widxv], rows, sem_w)
        cp_n = pltpu.async_copy(norm_hbm.at[keyv], normv, sem_n)
        cp_n.wait()
        cp_w.wait()

        def _scale(g, _):
            nv = normv[pl.ds(g * 16, 16)]
            for l in range(16):
                nb = lax.broadcast(nv[l], (16,))
                k = g * 16 + l
                for j in range(D // 16):
                    sl = pl.ds(j * 16, 16)
                    rows[k, sl] = rows[k, sl] * nb
            return 0
        lax.fori_loop(0, C // 16, _scale, 0)

        pltpu.sync_copy(rows, agg_sh.at[dstv], add=True)
        return 0
    lax.fori_loop(0, NCHUNK, _chunk, 0)

    plsc.subcore_barrier()

    def _wb(k, _):
        row0 = s * AGG_SLICE + k * C
        pltpu.sync_copy(agg_sh.at[pl.ds(row0, C)], out_hbm.at[c, pl.ds(row0, C)])
        return 0
    lax.fori_loop(0, AGG_SLICE // C, _wb, 0)


@functools.partial(
    pl.kernel,
    mesh=_mesh,
    out_type=jax.ShapeDtypeStruct((H_PAD, D), jnp.float32),
    scratch_types=[
        pltpu.VMEM((H_PER_W,), jnp.int32),
        pltpu.VMEM((H_PER_W, D), jnp.float32),
        pltpu.SemaphoreType.DMA,
    ],
)
def _sc_hgather(nf_hbm, ids_hbm, out_hbm, idsv, rows, sem):
    c = lax.axis_index("c")
    s = lax.axis_index("s")
    wid = s * NC + c
    off = wid * H_PER_W
    pltpu.sync_copy(ids_hbm.at[pl.ds(off, H_PER_W)], idsv)
    pltpu.async_copy(nf_hbm.at[idsv], rows, sem).wait()
    pltpu.sync_copy(rows, out_hbm.at[pl.ds(off, H_PER_W)])


def _tc_norm_body(cnt_ref, out_ref):
    csum = cnt_ref[0] + cnt_ref[1]
    out_ref[...] = 1.0 / jnp.maximum(csum, 1.0)


def _tc_norm(cnt2):
    cnt3 = cnt2.reshape(NC, KEYS // D, D)
    out = pl.pallas_call(
        _tc_norm_body,
        out_shape=jax.ShapeDtypeStruct((KEYS // D, D), jnp.float32),
    )(cnt3)
    return out.reshape(KEYS)


def _tc_weight_body(basis_ref, comp_ref, out_ref):
    out_ref[...] = jnp.dot(comp_ref[...], basis_ref[...],
                           preferred_element_type=jnp.float32)


def _tc_weight(basis, comp):
    basis2 = basis.reshape(NB, N * D)
    nblk = 40
    bn = (N * D) // nblk
    out = pl.pallas_call(
        _tc_weight_body,
        grid=(nblk,),
        in_specs=[
            pl.BlockSpec((NB, bn), lambda i: (0, i)),
            pl.BlockSpec((R, NB), lambda i: (0, 0)),
        ],
        out_specs=pl.BlockSpec((R, bn), lambda i: (0, i)),
        out_shape=jax.ShapeDtypeStruct((R, N * D), jnp.float32),
    )(basis2, comp)
    return out.reshape(R * N, D)


def _tc_feat_body(agg_ref, root_ref, bias_ref, out_ref):
    out_ref[...] = agg_ref[0] + agg_ref[1] + root_ref[...] + bias_ref[...]


def _tc_feat(agg2, root, rgcn_bias):
    nblk = 25
    bn = N // nblk
    bias2 = rgcn_bias.reshape(1, D)
    return pl.pallas_call(
        _tc_feat_body,
        grid=(nblk,),
        in_specs=[
            pl.BlockSpec((NC, bn, D), lambda i: (0, i, 0)),
            pl.BlockSpec((bn, D), lambda i: (i, 0)),
            pl.BlockSpec((1, D), lambda i: (0, 0)),
        ],
        out_specs=pl.BlockSpec((bn, D), lambda i: (i, 0)),
        out_shape=jax.ShapeDtypeStruct((N, D), jnp.float32),
    )(agg2, root, bias2)


def _tc_attn_body(h_ref, a_ref, b_ref, s_ref, out_ref):
    h = h_ref[...]
    t = jnp.tanh(jnp.dot(h, a_ref[...], preferred_element_type=jnp.float32))
    e = jnp.dot(t, b_ref[...], preferred_element_type=jnp.float32)
    p = jnp.exp(e - jnp.max(e))
    sel = s_ref[...]
    denom = jnp.dot(sel, p, preferred_element_type=jnp.float32)
    un = jnp.dot(sel, p * h, preferred_element_type=jnp.float32)
    out_ref[...] = un / denom


def _tc_attn(h, attn_a, attn_b, sel):
    return pl.pallas_call(
        _tc_attn_body,
        out_shape=jax.ShapeDtypeStruct((B, D), jnp.float32),
    )(h, attn_a, attn_b, sel)


def _tc_logits_body(u_ref, nf_ref, bias_ref, out_ref):
    out_ref[...] = lax.dot_general(
        u_ref[...], nf_ref[...], (((1,), (1,)), ((), ())),
        preferred_element_type=jnp.float32) + bias_ref[...]


def _tc_logits(u, nf, out_bias):
    bias2 = out_bias.reshape(1, N)
    return pl.pallas_call(
        _tc_logits_body,
        out_shape=jax.ShapeDtypeStruct((B, N), jnp.float32),
    )(u, nf, bias2)


def kernel(input_ids, attention_mask, edge_idx, edge_type, basis, comp, root,
           rgcn_bias, attn_a, attn_b, out_bias):
    src = edge_idx[0].astype(jnp.int32)
    dst = edge_idx[1].astype(jnp.int32)
    et = edge_type.astype(jnp.int32)

    npad = E_PAD - E
    pi = jnp.arange(npad, dtype=jnp.int32)
    src_p = jnp.concatenate([src, pi % N])
    dst_p = jnp.concatenate([dst, N + pi % (AGG_ROWS - N)])
    et_p = jnp.concatenate([et, jnp.zeros((npad,), jnp.int32)])

    cnt2 = _sc_count(dst_p, et_p)
    norm = _tc_norm(cnt2)
    wflat = _tc_weight(basis, comp)
    agg2 = _sc_main(src_p, dst_p, et_p, wflat, norm)
    nf = _tc_feat(agg2, root, rgcn_bias)

    ids = input_ids.astype(jnp.int32).reshape(B * L)
    hpad = H_PAD - B * L
    ids_p = jnp.concatenate([ids, (jnp.arange(hpad, dtype=jnp.int32) * 37) % N])
    h = _sc_hgather(nf, ids_p)

    sel = jnp.kron(jnp.eye(B, dtype=jnp.float32), jnp.ones((1, L), jnp.float32))
    sel = jnp.pad(sel, ((0, 0), (0, hpad)))
    u = _tc_attn(h, attn_a, attn_b, sel)
    return _tc_logits(u, nf, out_bias)


# speedup vs baseline: 10.3116x; 6.2745x over previous
import functools

import jax
import jax.numpy as jnp
from jax import lax
from jax.experimental import pallas as pl
from jax.experimental.pallas import tpu as pltpu
from jax.experimental.pallas import tpu_sc as plsc

N = 10000
R = 12
D = 128
NB = 8
E = 320000
B = 64
L = 50

NC = 2
NS = 16
NW = NC * NS

C = 64
T = 10240
E_PAD = NW * T
NCHUNK = T // C
PER_CORE = E_PAD // NC
ROWS_PER_CORE = PER_CORE // C

AGG_ROWS = N + 240
KEYS = AGG_ROWS * R
KEY_SLICE = KEYS // NS
AGG_SLICE = AGG_ROWS // NS

H_PAD = 3328
H_PER_W = H_PAD // NW

_mesh = plsc.VectorSubcoreMesh(core_axis_name="c", subcore_axis_name="s")


@functools.partial(
    pl.kernel,
    mesh=_mesh,
    out_type=jax.ShapeDtypeStruct((NC, KEYS), jnp.float32),
    scratch_types=[
        pltpu.VMEM((NCHUNK, C), jnp.int32),
        pltpu.VMEM((C,), jnp.float32),
        pltpu.VMEM((KEY_SLICE,), jnp.float32),
        pltpu.VMEM_SHARED((KEYS,), jnp.float32),
        pltpu.SemaphoreType.DMA,
    ],
)
def _sc_count(key_hbm, out_hbm, key2d, onesv, zerov, cnt_sh, sem):
    c = lax.axis_index("c")
    s = lax.axis_index("s")

    def _fill(i, _):
        zerov[pl.ds(i * 16, 16)] = jnp.zeros((16,), jnp.float32)
        return 0
    lax.fori_loop(0, KEY_SLICE // 16, _fill, 0)
    for j in range(C // 16):
        onesv[pl.ds(j * 16, 16)] = jnp.ones((16,), jnp.float32)
    pltpu.sync_copy(zerov, cnt_sh.at[pl.ds(s * KEY_SLICE, KEY_SLICE)])

    rowbase = c * ROWS_PER_CORE + s * NCHUNK
    pltpu.async_copy(key_hbm.at[pl.ds(rowbase, NCHUNK)], key2d, sem).wait()
    plsc.subcore_barrier()

    def _chunk(i, _):
        pltpu.sync_copy(onesv, cnt_sh.at[key2d.at[i]], add=True)
        return 0
    lax.fori_loop(0, NCHUNK, _chunk, 0)

    plsc.subcore_barrier()
    sl = pl.ds(s * KEY_SLICE, KEY_SLICE)
    pltpu.sync_copy(cnt_sh.at[sl], out_hbm.at[c, sl])


HALF = T // 2
NCH2 = NCHUNK // 2


@functools.partial(
    pl.kernel,
    mesh=_mesh,
    out_type=jax.ShapeDtypeStruct((NC, AGG_ROWS, D), jnp.float32),
    scratch_types=[
        pltpu.VMEM((HALF,), jnp.int32),
        pltpu.VMEM((HALF,), jnp.int32),
        pltpu.VMEM((NCH2, C), jnp.int32),
        pltpu.VMEM((C,), jnp.float32),
        pltpu.VMEM((C,), jnp.float32),
        pltpu.VMEM((C, D), jnp.float32),
        pltpu.VMEM((C, D), jnp.float32),
        pltpu.VMEM_SHARED((AGG_ROWS, D), jnp.float32),
        pltpu.SemaphoreType.DMA,
        pltpu.SemaphoreType.DMA,
        pltpu.SemaphoreType.DMA,
        pltpu.SemaphoreType.DMA,
    ],
)
def _sc_main(widx_hbm, key_hbm, dst_hbm, w_hbm, norm_hbm, out_hbm,
             widxb, keyb, dst2d, norm0, norm1,
             rows0, rows1, agg_sh, sem_w0, sem_w1, sem_n0, sem_n1):
    c = lax.axis_index("c")
    s = lax.axis_index("s")

    base = c * PER_CORE + s * T
    rowbase = c * ROWS_PER_CORE + s * NCHUNK
    cp_w = pltpu.async_copy(widx_hbm.at[pl.ds(base, HALF)], widxb, sem_w0)
    cp_k = pltpu.async_copy(key_hbm.at[pl.ds(base, HALF)], keyb, sem_w0)
    cp_d = pltpu.async_copy(dst_hbm.at[pl.ds(rowbase, NCH2)], dst2d, sem_w1)

    def _zrow(k, _):
        for j in range(D // 16):
            rows0[k, pl.ds(j * 16, 16)] = jnp.zeros((16,), jnp.float32)
        return 0
    lax.fori_loop(0, C, _zrow, 0)

    def _zcopy(k, _):
        pltpu.sync_copy(rows0, agg_sh.at[pl.ds(s * AGG_SLICE + k * C, C)])
        return 0
    lax.fori_loop(0, AGG_SLICE // C, _zcopy, 0)

    cp_w.wait()
    cp_k.wait()
    cp_d.wait()
    plsc.subcore_barrier()

    def _gather(i, rows, normv, sem_w, sem_n):
        cw = pltpu.async_copy(w_hbm.at[widxb.at[pl.ds(i * C, C)]], rows, sem_w)
        cn = pltpu.async_copy(norm_hbm.at[keyb.at[pl.ds(i * C, C)]], normv, sem_n)
        return cw, cn

    def _process(i, rows, normv):
        def _scale(g, _):
            nv = normv[pl.ds(g * 16, 16)]
            for l in range(16):
                nb = lax.broadcast(nv[l], (16,))
                k = g * 16 + l
                for j in range(D // 16):
                    sl = pl.ds(j * 16, 16)
                    rows[k, sl] = rows[k, sl] * nb
            return 0
        lax.fori_loop(0, C // 16, _scale, 0)
        pltpu.sync_copy(rows, agg_sh.at[dst2d.at[i]], add=True)

    def _half_pipeline():
        _gather(0, rows0, norm0, sem_w0, sem_n0)

        def _pair(i, _):
            a = 2 * i
            b = 2 * i + 1
            pltpu.make_async_copy(w_hbm.at[widxb.at[pl.ds(a * C, C)]], rows0, sem_w0).wait()
            pltpu.make_async_copy(norm_hbm.at[keyb.at[pl.ds(a * C, C)]], norm0, sem_n0).wait()
            _gather(b, rows1, norm1, sem_w1, sem_n1)
            _process(a, rows0, norm0)
            nxt = jnp.minimum(a + 2, NCH2 - 1)
            pltpu.make_async_copy(w_hbm.at[widxb.at[pl.ds(b * C, C)]], rows1, sem_w1).wait()
            pltpu.make_async_copy(norm_hbm.at[keyb.at[pl.ds(b * C, C)]], norm1, sem_n1).wait()
            _gather(nxt, rows0, norm0, sem_w0, sem_n0)
            _process(b, rows1, norm1)
            return 0
        lax.fori_loop(0, NCH2 // 2, _pair, 0)

        pltpu.make_async_copy(w_hbm.at[widxb.at[pl.ds(0, C)]], rows0, sem_w0).wait()
        pltpu.make_async_copy(norm_hbm.at[keyb.at[pl.ds(0, C)]], norm0, sem_n0).wait()

    _half_pipeline()

    pltpu.sync_copy(widx_hbm.at[pl.ds(base + HALF, HALF)], widxb)
    pltpu.sync_copy(key_hbm.at[pl.ds(base + HALF, HALF)], keyb)
    pltpu.sync_copy(dst_hbm.at[pl.ds(rowbase + NCH2, NCH2)], dst2d)
    _half_pipeline()

    plsc.subcore_barrier()

    def _wb(k, _):
        row0 = s * AGG_SLICE + k * C
        pltpu.sync_copy(agg_sh.at[pl.ds(row0, C)], out_hbm.at[c, pl.ds(row0, C)])
        return 0
    lax.fori_loop(0, AGG_SLICE // C, _wb, 0)


@functools.partial(
    pl.kernel,
    mesh=_mesh,
    out_type=jax.ShapeDtypeStruct((H_PAD, D), jnp.float32),
    scratch_types=[
        pltpu.VMEM((H_PER_W,), jnp.int32),
        pltpu.VMEM((H_PER_W, D), jnp.float32),
        pltpu.SemaphoreType.DMA,
    ],
)
def _sc_hgather(nf_hbm, ids_hbm, out_hbm, idsv, rows, sem):
    c = lax.axis_index("c")
    s = lax.axis_index("s")
    wid = s * NC + c
    off = wid * H_PER_W
    pltpu.sync_copy(ids_hbm.at[pl.ds(off, H_PER_W)], idsv)
    pltpu.async_copy(nf_hbm.at[idsv], rows, sem).wait()
    pltpu.sync_copy(rows, out_hbm.at[pl.ds(off, H_PER_W)])


def _tc_norm_body(cnt_ref, out_ref):
    csum = cnt_ref[0] + cnt_ref[1]
    out_ref[...] = 1.0 / jnp.maximum(csum, 1.0)


def _tc_norm(cnt2):
    cnt3 = cnt2.reshape(NC, KEYS // D, D)
    out = pl.pallas_call(
        _tc_norm_body,
        out_shape=jax.ShapeDtypeStruct((KEYS // D, D), jnp.float32),
    )(cnt3)
    return out.reshape(KEYS)


def _tc_weight_body(basis_ref, comp_ref, out_ref):
    for r in range(R):
        acc = comp_ref[r, 0] * basis_ref[0]
        for b in range(1, NB):
            acc = acc + comp_ref[r, b] * basis_ref[b]
        out_ref[r] = acc


def _tc_weight(basis, comp):
    nblk = 25
    bn = N // nblk
    out = pl.pallas_call(
        _tc_weight_body,
        grid=(nblk,),
        in_specs=[
            pl.BlockSpec((NB, bn, D), lambda i: (0, i, 0)),
            pl.BlockSpec((R, NB), lambda i: (0, 0)),
        ],
        out_specs=pl.BlockSpec((R, bn, D), lambda i: (0, i, 0)),
        out_shape=jax.ShapeDtypeStruct((R, N, D), jnp.float32),
    )(basis, comp)
    return out.reshape(R * N, D)


def _tc_feat_body(agg_ref, root_ref, bias_ref, out_ref):
    out_ref[...] = agg_ref[0] + agg_ref[1] + root_ref[...] + bias_ref[...]


def _tc_feat(agg2, root, rgcn_bias):
    nblk = 25
    bn = N // nblk
    bias2 = rgcn_bias.reshape(1, D)
    return pl.pallas_call(
        _tc_feat_body,
        grid=(nblk,),
        in_specs=[
            pl.BlockSpec((NC, bn, D), lambda i: (0, i, 0)),
            pl.BlockSpec((bn, D), lambda i: (i, 0)),
            pl.BlockSpec((1, D), lambda i: (0, 0)),
        ],
        out_specs=pl.BlockSpec((bn, D), lambda i: (i, 0)),
        out_shape=jax.ShapeDtypeStruct((N, D), jnp.float32),
    )(agg2, root, bias2)


def _tc_attn_body(h_ref, a_ref, b_ref, s_ref, out_ref):
    h = h_ref[...]
    t = jnp.tanh(jnp.dot(h, a_ref[...], preferred_element_type=jnp.float32))
    e = jnp.dot(t, b_ref[...], preferred_element_type=jnp.float32)
    p = jnp.exp(e - jnp.max(e))
    sel = s_ref[...]
    denom = jnp.dot(sel, p, preferred_element_type=jnp.float32)
    un = jnp.dot(sel, p * h, preferred_element_type=jnp.float32)
    out_ref[...] = un / denom


def _tc_attn(h, attn_a, attn_b, sel):
    return pl.pallas_call(
        _tc_attn_body,
        out_shape=jax.ShapeDtypeStruct((B, D), jnp.float32),
    )(h, attn_a, attn_b, sel)


def _tc_logits_body(u_ref, nf_ref, bias_ref, out_ref):
    out_ref[...] = lax.dot_general(
        u_ref[...], nf_ref[...], (((1,), (1,)), ((), ())),
        preferred_element_type=jnp.float32) + bias_ref[...]


def _tc_logits(u, nf, out_bias):
    bias2 = out_bias.reshape(1, N)
    return pl.pallas_call(
        _tc_logits_body,
        out_shape=jax.ShapeDtypeStruct((B, N), jnp.float32),
    )(u, nf, bias2)


def kernel(input_ids, attention_mask, edge_idx, edge_type, basis, comp, root,
           rgcn_bias, attn_a, attn_b, out_bias):
    src = edge_idx[0].astype(jnp.int32)
    dst = edge_idx[1].astype(jnp.int32)
    et = edge_type.astype(jnp.int32)

    npad = E_PAD - E
    pi = jnp.arange(npad, dtype=jnp.int32)
    src_p = jnp.concatenate([src, pi % N])
    dst_p = jnp.concatenate([dst, N + pi % (AGG_ROWS - N)])
    et_p = jnp.concatenate([et, jnp.zeros((npad,), jnp.int32)])
    widx_p = et_p * N + src_p
    key_p = dst_p * R + et_p

    cnt2 = _sc_count(key_p.reshape(E_PAD // C, C))
    norm = _tc_norm(cnt2)
    wflat = _tc_weight(basis, comp)
    agg2 = _sc_main(widx_p, key_p, dst_p.reshape(E_PAD // C, C), wflat, norm)
    nf = _tc_feat(agg2, root, rgcn_bias)

    ids = input_ids.astype(jnp.int32).reshape(B * L)
    hpad = H_PAD - B * L
    ids_p = jnp.concatenate([ids, (jnp.arange(hpad, dtype=jnp.int32) * 37) % N])
    h = _sc_hgather(nf, ids_p)

    sel = jnp.kron(jnp.eye(B, dtype=jnp.float32), jnp.ones((1, L), jnp.float32))
    sel = jnp.pad(sel, ((0, 0), (0, hpad)))
    u = _tc_attn(h, attn_a, attn_b, sel)
    return _tc_logits(u, nf, out_bias)


# speedup vs baseline: 10.3972x; 1.0083x over previous
import functools

import jax
import jax.numpy as jnp
from jax import lax
from jax.experimental import pallas as pl
from jax.experimental.pallas import tpu as pltpu
from jax.experimental.pallas import tpu_sc as plsc

N = 10000
R = 12
D = 128
NB = 8
E = 320000
B = 64
L = 50

NC = 2
NS = 16
NW = NC * NS

C = 64
T = 10240
E_PAD = NW * T
NCHUNK = T // C
PER_CORE = E_PAD // NC
ROWS_PER_CORE = PER_CORE // C

AGG_ROWS = N + 240
KEYS = AGG_ROWS * R
KEY_SLICE = KEYS // NS
AGG_SLICE = AGG_ROWS // NS

H_PAD = 3328
H_PER_W = H_PAD // NW

_mesh = plsc.VectorSubcoreMesh(core_axis_name="c", subcore_axis_name="s")


@functools.partial(
    pl.kernel,
    mesh=_mesh,
    out_type=jax.ShapeDtypeStruct((NC, KEYS), jnp.float32),
    scratch_types=[
        pltpu.VMEM((NCHUNK, C), jnp.int32),
        pltpu.VMEM((C,), jnp.float32),
        pltpu.VMEM((KEY_SLICE,), jnp.float32),
        pltpu.VMEM_SHARED((KEYS,), jnp.float32),
        pltpu.SemaphoreType.DMA,
    ],
)
def _sc_count(key_hbm, out_hbm, key2d, onesv, zerov, cnt_sh, sem):
    c = lax.axis_index("c")
    s = lax.axis_index("s")

    def _fill(i, _):
        zerov[pl.ds(i * 16, 16)] = jnp.zeros((16,), jnp.float32)
        return 0
    lax.fori_loop(0, KEY_SLICE // 16, _fill, 0)
    for j in range(C // 16):
        onesv[pl.ds(j * 16, 16)] = jnp.ones((16,), jnp.float32)
    pltpu.sync_copy(zerov, cnt_sh.at[pl.ds(s * KEY_SLICE, KEY_SLICE)])

    rowbase = c * ROWS_PER_CORE + s * NCHUNK
    pltpu.async_copy(key_hbm.at[pl.ds(rowbase, NCHUNK)], key2d, sem).wait()
    plsc.subcore_barrier()

    def _chunk(i, _):
        pltpu.sync_copy(onesv, cnt_sh.at[key2d.at[i]], add=True)
        return 0
    lax.fori_loop(0, NCHUNK, _chunk, 0)

    plsc.subcore_barrier()
    sl = pl.ds(s * KEY_SLICE, KEY_SLICE)
    pltpu.sync_copy(cnt_sh.at[sl], out_hbm.at[c, sl])


HALF = T // 2
NCH2 = NCHUNK // 2
H_PER_S = H_PAD // NS


@functools.partial(
    pl.kernel,
    mesh=_mesh,
    out_type=(
        jax.ShapeDtypeStruct((NC, AGG_ROWS, D), jnp.float32),
        jax.ShapeDtypeStruct((NC, H_PAD, D), jnp.float32),
    ),
    scratch_types=[
        pltpu.VMEM((HALF,), jnp.int32),
        pltpu.VMEM((HALF,), jnp.int32),
        pltpu.VMEM((NCH2, C), jnp.int32),
        pltpu.VMEM((C,), jnp.float32),
        pltpu.VMEM((C,), jnp.float32),
        pltpu.VMEM((C, D), jnp.float32),
        pltpu.VMEM((C, D), jnp.float32),
        pltpu.VMEM_SHARED((AGG_ROWS, D), jnp.float32),
        pltpu.SemaphoreType.DMA,
        pltpu.SemaphoreType.DMA,
        pltpu.SemaphoreType.DMA,
        pltpu.SemaphoreType.DMA,
    ],
)
def _sc_main(widx_hbm, key_hbm, dst_hbm, w_hbm, norm_hbm, root_hbm, ids_hbm,
             out_hbm, h_hbm,
             widxb, keyb, dst2d, norm0, norm1,
             rows0, rows1, agg_sh, sem_w0, sem_w1, sem_n0, sem_n1):
    c = lax.axis_index("c")
    s = lax.axis_index("s")

    base = c * PER_CORE + s * T
    rowbase = c * ROWS_PER_CORE + s * NCHUNK
    cp_w = pltpu.async_copy(widx_hbm.at[pl.ds(base, HALF)], widxb, sem_w0)
    cp_k = pltpu.async_copy(key_hbm.at[pl.ds(base, HALF)], keyb, sem_w0)
    cp_d = pltpu.async_copy(dst_hbm.at[pl.ds(rowbase, NCH2)], dst2d, sem_w1)

    @pl.when(c == 0)
    def _():
        pltpu.sync_copy(root_hbm.at[pl.ds(s * AGG_SLICE, AGG_SLICE)],
                        agg_sh.at[pl.ds(s * AGG_SLICE, AGG_SLICE)])

    @pl.when(c != 0)
    def _():
        def _zrow(k, _):
            for j in range(D // 16):
                rows0[k, pl.ds(j * 16, 16)] = jnp.zeros((16,), jnp.float32)
            return 0
        lax.fori_loop(0, C, _zrow, 0)

        def _zcopy(k, _):
            pltpu.sync_copy(rows0, agg_sh.at[pl.ds(s * AGG_SLICE + k * C, C)])
            return 0
        lax.fori_loop(0, AGG_SLICE // C, _zcopy, 0)

    cp_w.wait()
    cp_k.wait()
    cp_d.wait()
    plsc.subcore_barrier()

    def _gather(i, rows, normv, sem_w, sem_n):
        cw = pltpu.async_copy(w_hbm.at[widxb.at[pl.ds(i * C, C)]], rows, sem_w)
        cn = pltpu.async_copy(norm_hbm.at[keyb.at[pl.ds(i * C, C)]], normv, sem_n)
        return cw, cn

    def _process(i, rows, normv):
        def _scale(g, _):
            nv = normv[pl.ds(g * 16, 16)]
            for l in range(16):
                nb = lax.broadcast(nv[l], (16,))
                k = g * 16 + l
                for j in range(D // 16):
                    sl = pl.ds(j * 16, 16)
                    rows[k, sl] = rows[k, sl] * nb
            return 0
        lax.fori_loop(0, C // 16, _scale, 0)
        pltpu.sync_copy(rows, agg_sh.at[dst2d.at[i]], add=True)

    def _half_pipeline():
        _gather(0, rows0, norm0, sem_w0, sem_n0)

        def _pair(i, _):
            a = 2 * i
            b = 2 * i + 1
            pltpu.make_async_copy(w_hbm.at[widxb.at[pl.ds(a * C, C)]], rows0, sem_w0).wait()
            pltpu.make_async_copy(norm_hbm.at[keyb.at[pl.ds(a * C, C)]], norm0, sem_n0).wait()
            _gather(b, rows1, norm1, sem_w1, sem_n1)
            _process(a, rows0, norm0)
            nxt = jnp.minimum(a + 2, NCH2 - 1)
            pltpu.make_async_copy(w_hbm.at[widxb.at[pl.ds(b * C, C)]], rows1, sem_w1).wait()
            pltpu.make_async_copy(norm_hbm.at[keyb.at[pl.ds(b * C, C)]], norm1, sem_n1).wait()
            _gather(nxt, rows0, norm0, sem_w0, sem_n0)
            _process(b, rows1, norm1)
            return 0
        lax.fori_loop(0, NCH2 // 2, _pair, 0)

        pltpu.make_async_copy(w_hbm.at[widxb.at[pl.ds(0, C)]], rows0, sem_w0).wait()
        pltpu.make_async_copy(norm_hbm.at[keyb.at[pl.ds(0, C)]], norm0, sem_n0).wait()

    _half_pipeline()

    pltpu.sync_copy(widx_hbm.at[pl.ds(base + HALF, HALF)], widxb)
    pltpu.sync_copy(key_hbm.at[pl.ds(base + HALF, HALF)], keyb)
    pltpu.sync_copy(dst_hbm.at[pl.ds(rowbase + NCH2, NCH2)], dst2d)
    _half_pipeline()

    plsc.subcore_barrier()

    def _wb(k, _):
        row0 = s * AGG_SLICE + k * C
        pltpu.sync_copy(agg_sh.at[pl.ds(row0, C)], out_hbm.at[c, pl.ds(row0, C)])
        return 0
    lax.fori_loop(0, AGG_SLICE // C, _wb, 0)

    hoff = s * H_PER_S
    pltpu.sync_copy(ids_hbm.at[pl.ds(hoff, H_PER_S)], widxb.at[pl.ds(0, H_PER_S)])
    for k, hc in enumerate((C, C, C, H_PER_S - 3 * C)):
        buf = rows0 if k % 2 == 0 else rows1
        pltpu.sync_copy(agg_sh.at[widxb.at[pl.ds(k * C, hc)]],
                        buf.at[pl.ds(0, hc)])
        pltpu.sync_copy(buf.at[pl.ds(0, hc)],
                        h_hbm.at[c, pl.ds(hoff + k * C, hc)])


def _tc_norm_body(cnt_ref, out_ref):
    csum = cnt_ref[0] + cnt_ref[1]
    out_ref[...] = 1.0 / jnp.maximum(csum, 1.0)


def _tc_norm(cnt2):
    cnt3 = cnt2.reshape(NC, KEYS // D, D)
    out = pl.pallas_call(
        _tc_norm_body,
        out_shape=jax.ShapeDtypeStruct((KEYS // D, D), jnp.float32),
    )(cnt3)
    return out.reshape(KEYS)


def _tc_weight_body(basis_ref, comp_ref, out_ref):
    for r in range(R):
        acc = comp_ref[r, 0] * basis_ref[0]
        for b in range(1, NB):
            acc = acc + comp_ref[r, b] * basis_ref[b]
        out_ref[r] = acc


def _tc_weight(basis, comp):
    nblk = 25
    bn = N // nblk
    out = pl.pallas_call(
        _tc_weight_body,
        grid=(nblk,),
        in_specs=[
            pl.BlockSpec((NB, bn, D), lambda i: (0, i, 0)),
            pl.BlockSpec((R, NB), lambda i: (0, 0)),
        ],
        out_specs=pl.BlockSpec((R, bn, D), lambda i: (0, i, 0)),
        out_shape=jax.ShapeDtypeStruct((R, N, D), jnp.float32),
    )(basis, comp)
    return out.reshape(R * N, D)


def _tc_attn_body(h_ref, a_ref, b_ref, s_ref, out_ref):
    h = h_ref[0] + h_ref[1]
    t = jnp.tanh(jnp.dot(h, a_ref[...], preferred_element_type=jnp.float32))
    e = jnp.dot(t, b_ref[...], preferred_element_type=jnp.float32)
    p = jnp.exp(e - jnp.max(e))
    sel = s_ref[...]
    denom = jnp.dot(sel, p, preferred_element_type=jnp.float32)
    un = jnp.dot(sel, p * h, preferred_element_type=jnp.float32)
    out_ref[...] = un / denom


def _tc_attn(h, attn_a, attn_b, sel):
    return pl.pallas_call(
        _tc_attn_body,
        out_shape=jax.ShapeDtypeStruct((B, D), jnp.float32),
    )(h, attn_a, attn_b, sel)


def _tc_logits_body(u_ref, agg_ref, bias_ref, out_ref):
    nf = agg_ref[0] + agg_ref[1]
    out_ref[...] = lax.dot_general(
        u_ref[...], nf, (((1,), (1,)), ((), ())),
        preferred_element_type=jnp.float32) + bias_ref[...]


def _tc_logits(u, agg2, out_bias):
    nblk = 10
    bn = AGG_ROWS // nblk
    bias2 = jnp.pad(out_bias, (0, AGG_ROWS - N)).reshape(1, AGG_ROWS)
    out = pl.pallas_call(
        _tc_logits_body,
        grid=(nblk,),
        in_specs=[
            pl.BlockSpec((B, D), lambda i: (0, 0)),
            pl.BlockSpec((NC, bn, D), lambda i: (0, i, 0)),
            pl.BlockSpec((1, bn), lambda i: (0, i)),
        ],
        out_specs=pl.BlockSpec((B, bn), lambda i: (0, i)),
        out_shape=jax.ShapeDtypeStruct((B, AGG_ROWS), jnp.float32),
    )(u, agg2, bias2)
    return out[:, :N]


def kernel(input_ids, attention_mask, edge_idx, edge_type, basis, comp, root,
           rgcn_bias, attn_a, attn_b, out_bias):
    src = edge_idx[0].astype(jnp.int32)
    dst = edge_idx[1].astype(jnp.int32)
    et = edge_type.astype(jnp.int32)

    npad = E_PAD - E
    pi = jnp.arange(npad, dtype=jnp.int32)
    src_p = jnp.concatenate([src, pi % N])
    dst_p = jnp.concatenate([dst, N + pi % (AGG_ROWS - N)])
    et_p = jnp.concatenate([et, jnp.zeros((npad,), jnp.int32)])
    widx_p = et_p * N + src_p
    key_p = dst_p * R + et_p

    cnt2 = _sc_count(key_p.reshape(E_PAD // C, C))
    norm = _tc_norm(cnt2)
    wflat = _tc_weight(basis, comp)

    root_pad = jnp.pad(root + rgcn_bias.reshape(1, D),
                       ((0, AGG_ROWS - N), (0, 0)))
    ids = input_ids.astype(jnp.int32).reshape(B * L)
    hpad = H_PAD - B * L
    ids_p = jnp.concatenate([ids, (jnp.arange(hpad, dtype=jnp.int32) * 37) % N])

    agg2, h2 = _sc_main(widx_p, key_p, dst_p.reshape(E_PAD // C, C),
                        wflat, norm, root_pad, ids_p)

    sel = jnp.kron(jnp.eye(B, dtype=jnp.float32), jnp.ones((1, L), jnp.float32))
    sel = jnp.pad(sel, ((0, 0), (0, hpad)))
    u = _tc_attn(h2, attn_a, attn_b, sel)
    return _tc_logits(u, agg2, out_bias)
